# baseline (speedup 1.0000x reference)
import functools

import jax
import jax.numpy as jnp
from jax import lax
from jax.experimental import pallas as pl
from jax.experimental.pallas import tpu as pltpu
from jax.experimental.pallas import tpu_sc as plsc

BATCH = 16384
SEQ = 10
EMBED_DIM = 64
VOCAB = 1000

_NC = 2
_NS = 16
_NW = _NC * _NS
_ROWS = BATCH * SEQ
_RPW = _ROWS // _NW
_CHUNK = 128
_NCH = _RPW // _CHUNK
_NBUF = 4
_L = 16


@functools.partial(
    pl.kernel,
    mesh=plsc.VectorSubcoreMesh(core_axis_name="c", subcore_axis_name="s"),
    out_type=jax.ShapeDtypeStruct((_ROWS * EMBED_DIM,), jnp.float32),
    scratch_types=[
        pltpu.VMEM((VOCAB * EMBED_DIM,), jnp.float32),
        pltpu.VMEM((_RPW,), jnp.int32),
        pltpu.VMEM((_NBUF, _CHUNK * EMBED_DIM), jnp.float32),
        pltpu.SemaphoreType.DMA((_NBUF,)),
    ],
    compiler_params=pltpu.CompilerParams(
        use_tc_tiling_on_sc=False, needs_layout_passes=False
    ),
)
def _embedding_rows(idx_hbm, table_hbm, out_hbm, table_v, idx_v, stage_v,
                    wsem):
    wid = lax.axis_index("s") * _NC + lax.axis_index("c")
    r0 = wid * _RPW

    pltpu.sync_copy(table_hbm, table_v)
    pltpu.sync_copy(idx_hbm.at[pl.ds(r0, _RPW)], idx_v)

    siota = lax.iota(jnp.int32, _L) * EMBED_DIM

    def fill(c, buf):
        @plsc.parallel_loop(0, _CHUNK // _L, unroll=2)
        def _group(g):
            iv = idx_v[pl.ds(c * _CHUNK + g * _L, _L)]
            src0 = iv * EMBED_DIM
            dst0 = siota + g * (_L * EMBED_DIM)
            for e in range(EMBED_DIM):
                vals = plsc.load_gather(table_v, [src0 + e])
                plsc.store_scatter(stage_v.at[buf], [dst0 + e], vals)

    def start_write(c, buf):
        pltpu.async_copy(
            stage_v.at[buf],
            out_hbm.at[pl.ds((r0 + c * _CHUNK) * EMBED_DIM,
                             _CHUNK * EMBED_DIM)],
            wsem.at[buf])

    def wait_write(c, buf):
        pltpu.make_async_copy(
            stage_v.at[buf],
            out_hbm.at[pl.ds((r0 + c * _CHUNK) * EMBED_DIM,
                             _CHUNK * EMBED_DIM)],
            wsem.at[buf]).wait()

    @pl.loop(0, _NCH)
    def _chunk(c):
        buf = c & (_NBUF - 1)

        @pl.when(c >= _NBUF)
        def _():
            wait_write(c - _NBUF, buf)

        fill(c, buf)
        start_write(c, buf)

    for c in range(_NCH - _NBUF, _NCH):
        wait_write(c, c % _NBUF)


def kernel(inputs, table):
    idx1 = inputs.reshape(_ROWS)
    table1 = table.reshape(VOCAB * EMBED_DIM)
    out = _embedding_rows(idx1, table1)
    return out.reshape(BATCH, SEQ, EMBED_DIM)


# speedup vs baseline: 2.7115x; 2.7115x over previous
import functools

import jax
import jax.numpy as jnp
from jax import lax
from jax.experimental import pallas as pl
from jax.experimental.pallas import tpu as pltpu
from jax.experimental.pallas import tpu_sc as plsc

BATCH = 16384
SEQ = 10
EMBED_DIM = 64
VOCAB = 1000

_NC = 2
_NS = 16
_NW = _NC * _NS
_ROWS = BATCH * SEQ
_RPW = _ROWS // _NW
_CHUNK = 128
_NCH = _RPW // _CHUNK
_NBUF = 4
_LAG = 2


@functools.partial(
    pl.kernel,
    mesh=plsc.VectorSubcoreMesh(core_axis_name="c", subcore_axis_name="s"),
    out_type=jax.ShapeDtypeStruct((_ROWS, EMBED_DIM), jnp.float32),
    scratch_types=[
        pltpu.VMEM((_NCH, _CHUNK), jnp.int32),
        pltpu.VMEM((_NBUF, _CHUNK, EMBED_DIM), jnp.float32),
        pltpu.VMEM_SHARED((VOCAB, EMBED_DIM), jnp.float32),
        pltpu.SemaphoreType.DMA((_NBUF,)),
        pltpu.SemaphoreType.DMA((_NBUF,)),
    ],
    compiler_params=pltpu.CompilerParams(use_tc_tiling_on_sc=False),
)
def _embedding_rows(idx_hbm, table_hbm, out_hbm, idx_v, rows_v, table_v,
                    gsem, wsem):
    wid = lax.axis_index("s") * _NC + lax.axis_index("c")
    r0 = wid * _RPW

    @pl.when(lax.axis_index("s") == 0)
    def _():
        pltpu.sync_copy(table_hbm, table_v)

    pltpu.sync_copy(idx_hbm.at[pl.ds(wid * _NCH, _NCH)], idx_v)
    plsc.subcore_barrier()

    def start_gather(c, buf):
        pltpu.async_copy(table_v.at[idx_v.at[c]], rows_v.at[buf],
                         gsem.at[buf])

    def wait_gather(c, buf):
        pltpu.make_async_copy(table_v.at[idx_v.at[c]], rows_v.at[buf],
                              gsem.at[buf]).wait()

    def start_write(c, buf):
        pltpu.async_copy(rows_v.at[buf],
                         out_hbm.at[pl.ds(r0 + c * _CHUNK, _CHUNK)],
                         wsem.at[buf])

    def wait_write(c, buf):
        pltpu.make_async_copy(rows_v.at[buf],
                              out_hbm.at[pl.ds(r0 + c * _CHUNK, _CHUNK)],
                              wsem.at[buf]).wait()

    @pl.loop(0, _NCH)
    def _chunk(c):
        for buf in range(_NBUF):

            @pl.when((c & (_NBUF - 1)) == buf)
            def _():
                @pl.when(c >= _NBUF)
                def _():
                    wait_write(c - _NBUF, buf)

                start_gather(c, buf)

                wbuf = (buf + _NBUF - _LAG) % _NBUF

                @pl.when(c >= _LAG)
                def _():
                    wait_gather(c - _LAG, wbuf)
                    start_write(c - _LAG, wbuf)

    for c in range(_NCH - _LAG, _NCH):
        wait_gather(c, c % _NBUF)
        start_write(c, c % _NBUF)
    for c in range(_NCH - _NBUF, _NCH):
        wait_write(c, c % _NBUF)


def kernel(inputs, table):
    idx2 = inputs.reshape(_NW * _NCH, _CHUNK)
    out = _embedding_rows(idx2, table)
    return out.reshape(BATCH, SEQ, EMBED_DIM)
